# baseline (speedup 1.0000x reference)
import functools

import jax
import jax.numpy as jnp
from jax.experimental import pallas as pl
from jax.experimental.pallas import tpu as pltpu

_NUM_GRAPHS = 64


def _edge_softmax(logits, dst, n):
    m = jax.ops.segment_max(logits, dst, num_segments=n)
    m = jnp.where(jnp.isfinite(m), m, 0.0)
    e = jnp.exp(logits - m[dst])
    s = jax.ops.segment_sum(e, dst, num_segments=n)
    return e / (s[dst] + 1e-16)


def _gat_conv(x, src, dst, W, a_s, a_d, b, H, C):
    N = x.shape[0]
    h = (x @ W).reshape(N, H, C)
    alpha_src = jnp.sum(h * a_s[None, :, :], axis=-1)
    alpha_dst = jnp.sum(h * a_d[None, :, :], axis=-1)
    logits = jax.nn.leaky_relu(alpha_src[src] + alpha_dst[dst], 0.2)
    alpha = _edge_softmax(logits, dst, N)
    out = jax.ops.segment_sum(h[src] * alpha[:, :, None], dst, num_segments=N)
    return out.mean(axis=1) + b


def _gcn_conv(x, src, dst, W, b):
    N = x.shape[0]
    loop = jnp.arange(N)
    s2 = jnp.concatenate([src, loop])
    d2 = jnp.concatenate([dst, loop])
    deg = jax.ops.segment_sum(jnp.ones(s2.shape[0], x.dtype), d2, num_segments=N)
    dinv = jax.lax.rsqrt(jnp.maximum(deg, 1.0))
    h = x @ W
    norm = (dinv[s2] * dinv[d2])[:, None]
    out = jax.ops.segment_sum(h[s2] * norm, d2, num_segments=N)
    return out + b


def _transformer_conv(x, src, dst, Wq, Wk, Wv, Wskip, b, H, C):
    N = x.shape[0]
    q = (x @ Wq).reshape(N, H, C)
    k = (x @ Wk).reshape(N, H, C)
    v = (x @ Wv).reshape(N, H, C)
    logits = jnp.sum(q[dst] * k[src], axis=-1) / jnp.sqrt(jnp.asarray(C, x.dtype))
    alpha = _edge_softmax(logits, dst, N)
    agg = jax.ops.segment_sum(v[src] * alpha[:, :, None], dst, num_segments=N)
    return agg.mean(axis=1) + x @ Wskip + b


def _gap(x, batch, G):
    s = jax.ops.segment_sum(x, batch, num_segments=G)
    c = jax.ops.segment_sum(jnp.ones(x.shape[0], x.dtype), batch, num_segments=G)
    return s / jnp.maximum(c, 1.0)[:, None]


def _head_body(xg_ref, xt_ref, fin_ref, wfc1_ref, bfc1_ref, wfc2_ref, bfc2_ref,
               wb1g_ref, wb1t_ref, wb1f_ref, bb1_ref, wb2_ref, bb2_ref,
               wb3_ref, bb3_ref, wb4_ref, bb4_ref, out_ref):
    f32 = jnp.float32
    fpn = jnp.maximum(jnp.dot(fin_ref[...], wfc1_ref[...],
                              preferred_element_type=f32) + bfc1_ref[...], 0.0)
    fpn = jnp.maximum(jnp.dot(fpn, wfc2_ref[...],
                              preferred_element_type=f32) + bfc2_ref[...], 0.0)
    z = (jnp.dot(xg_ref[...], wb1g_ref[...], preferred_element_type=f32)
         + jnp.dot(xt_ref[...], wb1t_ref[...], preferred_element_type=f32)
         + jnp.dot(fpn, wb1f_ref[...], preferred_element_type=f32)
         + bb1_ref[...])
    z = jnp.maximum(z, 0.0)
    z = jnp.maximum(jnp.dot(z, wb2_ref[...], preferred_element_type=f32)
                    + bb2_ref[...], 0.0)
    z = jnp.maximum(jnp.dot(z, wb3_ref[...], preferred_element_type=f32)
                    + bb3_ref[...], 0.0)
    y = jax.nn.sigmoid(jnp.dot(z, wb4_ref[...], preferred_element_type=f32)
                       + bb4_ref[...])
    out_ref[...] = y


def _head(x_gat, x_trans, finger, Wfc1, bfc1, Wfc2, bfc2,
          Wb1, bb1, Wb2, bb2, Wb3, bb3, Wb4, bb4):
    G = x_gat.shape[0]
    wb1g = Wb1[:256]
    wb1t = Wb1[256:768]
    wb1f = Wb1[768:]
    out = pl.pallas_call(
        _head_body,
        out_shape=jax.ShapeDtypeStruct((G, 2), jnp.float32),
    )(x_gat, x_trans, finger, Wfc1, bfc1.reshape(1, -1), Wfc2,
      bfc2.reshape(1, -1), wb1g, wb1t, wb1f, bb1.reshape(1, -1),
      Wb2, bb2.reshape(1, -1), Wb3, bb3.reshape(1, -1), Wb4,
      bb4.reshape(1, -1))
    return out


def kernel(x, finger, edge_index, batch, W1, as1, ad1, b1, W2, as2, ad2, b2,
           W3, as3, ad3, b3, W4, b4, Wq, Wk, Wv, Wskip, bt, Wfc1, bfc1,
           Wfc2, bfc2, Wb1, bb1, Wb2, bb2, Wb3, bb3, Wb4, bb4):
    src, dst = edge_index[0], edge_index[1]
    G = _NUM_GRAPHS
    h = jax.nn.relu(_gat_conv(x, src, dst, W1, as1, ad1, b1, 8, 2048))
    h = jax.nn.relu(_gat_conv(h, src, dst, W2, as2, ad2, b2, 8, 1024))
    h = jax.nn.relu(_gat_conv(h, src, dst, W3, as3, ad3, b3, 8, 512))
    h = _gcn_conv(h, src, dst, W4, b4)
    x_gat = _gap(h, batch, G)
    xt = jax.nn.relu(_transformer_conv(x, src, dst, Wq, Wk, Wv, Wskip, bt, 4, 512))
    x_trans = _gap(xt, batch, G)
    return _head(x_gat, x_trans, finger, Wfc1, bfc1, Wfc2, bfc2,
                 Wb1, bb1, Wb2, bb2, Wb3, bb3, Wb4, bb4)
